# Initial kernel scaffold; baseline (speedup 1.0000x reference)
import jax
import jax.numpy as jnp
from jax.experimental import pallas as pl


def kernel(token_ids, weight):
    raise NotImplementedError("write your pallas kernel here")


# speedup vs baseline: 1.6826x; 1.6826x over previous
import functools

import jax
import jax.numpy as jnp
from jax import lax
from jax.experimental import pallas as pl
from jax.experimental.pallas import tpu as pltpu
from jax.experimental.pallas import tpu_sc as plsc

D_MODEL = 64
GROUP = 128


@functools.partial(jax.jit, static_argnums=(2, 3))
def _sc_embedding_gather(idx_grouped, weight, ngroups, group):
    nw = idx_grouped.shape[0]
    b_total = nw * ngroups * group
    mesh = plsc.VectorSubcoreMesh(core_axis_name="c", subcore_axis_name="s")
    nc = plsc.get_sparse_core_info().num_cores

    @functools.partial(
        pl.kernel,
        mesh=mesh,
        out_type=jax.ShapeDtypeStruct((b_total, D_MODEL), jnp.float32),
        scratch_types=[
            pltpu.VMEM((ngroups, group), jnp.int32),
            pltpu.VMEM((group, D_MODEL), jnp.float32),
            pltpu.SemaphoreType.DMA,
        ],
        compiler_params=pltpu.CompilerParams(use_tc_tiling_on_sc=False),
    )
    def k(weight_hbm, idx_hbm, out_hbm, idx_v, rows_v, gsem):
        wid = lax.axis_index("s") * nc + lax.axis_index("c")
        base = wid * (ngroups * group)
        pltpu.sync_copy(idx_hbm.at[wid], idx_v)

        def body(g, _):
            pltpu.async_copy(weight_hbm.at[idx_v.at[g]], rows_v, gsem).wait()
            pltpu.sync_copy(rows_v, out_hbm.at[pl.ds(base + g * group, group)])
            return 0

        lax.fori_loop(0, ngroups, body, 0)

    return k(weight, idx_grouped)


def kernel(token_ids, weight):
    b, h = token_ids.shape
    total = b * h
    nw = 32
    assert total % (nw * GROUP) == 0
    ngroups = total // (nw * GROUP)
    idx = token_ids.astype(jnp.int32).reshape(nw, ngroups, GROUP)
    out = _sc_embedding_gather(idx, weight, ngroups, GROUP)
    return out.reshape(b, h, D_MODEL)


# speedup vs baseline: 1.8766x; 1.1153x over previous
import functools

import jax
import jax.numpy as jnp
from jax import lax
from jax.experimental import pallas as pl
from jax.experimental.pallas import tpu as pltpu
from jax.experimental.pallas import tpu_sc as plsc

D_MODEL = 64
GROUP = 128
NBUF = 8
K = 4


@functools.partial(jax.jit, static_argnums=(2, 3))
def _sc_embedding_gather(idx_grouped, weight, ngroups, group):
    nw = idx_grouped.shape[0]
    b_total = nw * ngroups * group
    mesh = plsc.VectorSubcoreMesh(core_axis_name="c", subcore_axis_name="s")
    nc = plsc.get_sparse_core_info().num_cores
    assert ngroups % NBUF == 0 and K < NBUF

    @functools.partial(
        pl.kernel,
        mesh=mesh,
        out_type=jax.ShapeDtypeStruct((b_total, D_MODEL), jnp.float32),
        scratch_types=[
            pltpu.VMEM((ngroups, group), jnp.int32),
            pltpu.VMEM((NBUF, group, D_MODEL), jnp.float32),
            pltpu.SemaphoreType.DMA((NBUF,)),
            pltpu.SemaphoreType.DMA((NBUF,)),
        ],
        compiler_params=pltpu.CompilerParams(use_tc_tiling_on_sc=False),
    )
    def k(weight_hbm, idx_hbm, out_hbm, idx_v, rows_v, gsem, ssem):
        wid = lax.axis_index("s") * nc + lax.axis_index("c")
        base = wid * (ngroups * group)
        pltpu.sync_copy(idx_hbm.at[wid], idx_v)

        def gather_start(g, b):
            pltpu.async_copy(weight_hbm.at[idx_v.at[g]], rows_v.at[b],
                             gsem.at[b])

        def gather_wait(g, b):
            pltpu.make_async_copy(weight_hbm.at[idx_v.at[g]], rows_v.at[b],
                                  gsem.at[b]).wait()

        def scat_start(g, b):
            pltpu.async_copy(rows_v.at[b],
                             out_hbm.at[pl.ds(base + g * group, group)],
                             ssem.at[b])

        def scat_wait(g, b):
            pltpu.make_async_copy(rows_v.at[b],
                                  out_hbm.at[pl.ds(base + g * group, group)],
                                  ssem.at[b]).wait()

        for b in range(K):
            gather_start(b, b)

        def outer(t, _):
            for j in range(NBUF):
                g = t * NBUF + j
                gather_wait(g, j)
                scat_start(g, j)
                gn = g + K
                bn = (j + K) % NBUF

                @pl.when(gn < ngroups)
                def _():
                    @pl.when(gn >= NBUF)
                    def _():
                        scat_wait(gn - NBUF, bn)

                    gather_start(gn, bn)

            return 0

        lax.fori_loop(0, ngroups // NBUF, outer, 0)
        for b in range(NBUF):
            scat_wait(ngroups - NBUF + b, b)

    return k(weight, idx_grouped)


def kernel(token_ids, weight):
    b, h = token_ids.shape
    total = b * h
    nw = 32
    assert total % (nw * GROUP) == 0
    ngroups = total // (nw * GROUP)
    idx = token_ids.astype(jnp.int32).reshape(nw, ngroups, GROUP)
    out = _sc_embedding_gather(idx, weight, ngroups, GROUP)
    return out.reshape(b, h, D_MODEL)
